# baseline (speedup 1.0000x reference)
import functools

import jax
import jax.numpy as jnp
from jax import lax
from jax.experimental import pallas as pl
from jax.experimental.pallas import tpu as pltpu
from jax.experimental.pallas import tpu_sc as plsc

N = 50000
D = 256
B = 256
N_ITERS = 6

NUM_WORKERS = 32
SEG_PER_W = B // NUM_WORKERS
R = 128
SCAP = N + SEG_PER_W * 16 + 16
NCHUNK = D // 16


def _lstm_body(h_ref, c_ref, r_ref, wq_ref, wr_ref, b_ref, h_out, c_out):
    gates = (jnp.dot(h_ref[...], wq_ref[...], preferred_element_type=jnp.float32)
             + jnp.dot(r_ref[...], wr_ref[...], preferred_element_type=jnp.float32)
             + b_ref[...])
    i_g = jax.nn.sigmoid(gates[:, 0:D])
    f_g = jax.nn.sigmoid(gates[:, D:2 * D])
    g_g = jnp.tanh(gates[:, 2 * D:3 * D])
    o_g = jax.nn.sigmoid(gates[:, 3 * D:4 * D])
    c_new = f_g * c_ref[...] + i_g * g_g
    c_out[...] = c_new
    h_out[...] = o_g * jnp.tanh(c_new)


def _attn_body(feat_hbm, ss_hbm, q_hbm, out_hbm,
               scores_v, fbuf, fbuf2, q_own, racc, ss_v, m_vec, sinv_vec,
               hs0, hsm, ss_s, poff_s, sem_a, sem_b):
    wid = lax.axis_index("c") * 16 + lax.axis_index("s")
    seg0 = pl.multiple_of(wid * SEG_PER_W, 8)

    pltpu.sync_copy(ss_hbm.at[pl.ds(seg0, 16)], ss_v)
    pltpu.sync_copy(q_hbm.at[pl.ds(seg0, SEG_PER_W)], q_own)

    ss_vec = ss_v[pl.ds(0, 16)]
    for j in range(SEG_PER_W + 1):
        ss_s[j] = ss_vec[j]
    range_start = ss_s[0]
    range_end = ss_s[SEG_PER_W]
    ss_s[SEG_PER_W + 1] = range_end
    ss_s[SEG_PER_W + 2] = range_end

    zero16 = jnp.zeros((16,), jnp.float32)
    ninf16 = jnp.full((16,), -jnp.inf, jnp.float32)
    lane = lax.iota(jnp.int32, 16)

    for w in range(0, 144, 16):
        hs0[pl.ds(w, 16)] = zero16
    hsm[pl.ds(0, 16)] = ninf16
    hsm[pl.ds(32, 16)] = ninf16
    scores_v[pl.ds(SCAP - 16, 16)] = zero16
    for j in range(SEG_PER_W):
        m_vec[j, pl.ds(0, 16)] = ninf16
        for k in range(NCHUNK):
            racc[j, pl.ds(k * 16, 16)] = zero16


    def hsum_at(v, t, base=16):
        for step in (8, 4, 2, 1):
            hs0[pl.ds(base, 16)] = v
            v = v + hs0[pl.ds(base - step, 16)]
        hs0[pl.ds(base, 16)] = v
        return hs0[pl.ds(base + 15 - t, 16)]

    def bcast0(v, base=16):
        for step in (1, 2, 4, 8):
            hs0[pl.ds(base, 16)] = v
            v = v + hs0[pl.ds(base - step, 16)]
        return v

    def hsum_bcast(v):
        for step in (8, 4, 2, 1):
            hs0[pl.ds(16, 16)] = v
            v = v + hs0[pl.ds(16 - step, 16)]
        v = jnp.where(lane == 15, v, 0.0)
        for step in (1, 2, 4, 8):
            hs0[pl.ds(16, 16)] = v
            v = v + hs0[pl.ds(16 + step, 16)]
        return v

    def hmax_bcast(v):
        for step in (8, 4, 2, 1):
            hsm[pl.ds(16, 16)] = v
            v = jnp.maximum(v, hsm[pl.ds(16 - step, 16)])
        v = jnp.where(lane == 15, v, -jnp.inf)
        for step in (1, 2, 4, 8):
            hsm[pl.ds(16, 16)] = v
            v = jnp.maximum(v, hsm[pl.ds(16 + step, 16)])
        return v

    po = jnp.int32(0)
    for j in range(SEG_PER_W):
        poff_s[j] = po
        seg_len = ss_s[j + 1] - ss_s[j]
        n_groups = (seg_len + 15) // 16
        scores_v[pl.ds(po + jnp.maximum(n_groups - 1, 0) * 16, 16)] = zero16
        po = po + n_groups * 16

    bs0 = (range_start // 8) * 8
    nblk = (range_end - bs0 + R - 1) // R

    def blk_start(bg):
        return pl.multiple_of(
            (jnp.minimum(bs0 + bg * R, N - R) // 8) * 8, 8)

    def dma_start(bg, buf, sem):
        pltpu.make_async_copy(feat_hbm.at[pl.ds(blk_start(bg), R)],
                              buf, sem).start()

    def dma_wait(buf, sem):
        pltpu.make_async_copy(feat_hbm.at[pl.ds(0, R)], buf, sem).wait()

    def run_sweep(process_block, state0):
        dma_start(0, fbuf, sem_a)
        dma_start(1, fbuf2, sem_b)

        def pair_body(gp, st):
            for i, (buf, sem) in enumerate(((fbuf, sem_a), (fbuf2, sem_b))):
                g_blk = gp * 2 + i
                dma_wait(buf, sem)
                st = process_block(g_blk, buf, st)
                dma_start(g_blk + 2, buf, sem)
            return st

        st = lax.fori_loop(0, (nblk + 1) // 2, pair_body, state0)
        dma_wait(fbuf, sem_a)
        dma_wait(fbuf2, sem_b)
        return st

    def s1_block(g_blk, fbuf, state):
        pos, sj = state
        bs = blk_start(g_blk)
        pe = jnp.minimum(bs + R, range_end)

        def run_body(_, st):
            p, sj_ = st
            a_j = ss_s[sj_]
            run_end = jnp.minimum(ss_s[sj_ + 1], pe)
            po_j = poff_s[sj_]
            sjc = jnp.minimum(sj_, SEG_PER_W - 1)
            qrow = [q_own[sjc, pl.ds(k * 16, 16)] for k in range(NCHUNK)]
            mv0 = m_vec[sjc, pl.ds(0, 16)]

            g0 = (p - a_j) // 16
            g1 = (run_end - a_j + 15) // 16

            def grp_body(g, m_acc):
                goff = po_j + g * 16
                gvec0 = scores_v[pl.ds(goff, 16)]
                gbase = a_j + g * 16
                lo = jnp.maximum(p, gbase)
                hi = jnp.minimum(run_end, gbase + 16)

                def node_quad(t2, gvec):
                    base_n = lo + 4 * t2

                    def dot(n, base):
                        row = n - bs
                        acc = [fbuf[row, pl.ds(k * 16, 16)] * qrow[k]
                               for k in range(4)]
                        for k in range(4, NCHUNK):
                            acc[k % 4] = (acc[k % 4] + fbuf[row, pl.ds(k * 16, 16)]
                                          * qrow[k])
                        return hsum_at((acc[0] + acc[1]) + (acc[2] + acc[3]),
                                       n - gbase, base)

                    ns = [jnp.minimum(base_n + u, hi - 1) for u in range(4)]
                    sbs = [dot(ns[u], 16 + 32 * u) for u in range(4)]
                    for u in range(4):
                        gvec = jnp.where(lane == (ns[u] - gbase), sbs[u], gvec)
                    return gvec

                gvec1 = lax.fori_loop(0, (hi - lo + 3) // 4, node_quad, gvec0)
                scores_v[pl.ds(goff, 16)] = gvec1
                valid = lane < (hi - gbase)
                return jnp.maximum(m_acc, jnp.where(valid, gvec1, ninf16))

            m_fin = lax.fori_loop(g0, g1, grp_body, mv0)
            m_vec[sjc, pl.ds(0, 16)] = m_fin
            adv = ss_s[sj_ + 1] <= pe
            sj_next = jnp.where(adv, jnp.minimum(sj_ + 1, SEG_PER_W), sj_)
            return jnp.maximum(p, run_end), sj_next

        return lax.fori_loop(0, SEG_PER_W, run_body, (pos, sj))

    run_sweep(s1_block, (range_start, jnp.int32(0)))

    for j in range(SEG_PER_W):
        seg_len = ss_s[j + 1] - ss_s[j]
        po_j = poff_s[j]
        mb = hmax_bcast(m_vec[j, pl.ds(0, 16)])

        def g_body(g, s_acc, po_j=po_j, mb=mb):
            off = po_j + g * 16
            ex = jnp.exp(jnp.minimum(scores_v[pl.ds(off, 16)] - mb, 0.0))
            scores_v[pl.ds(off, 16)] = ex
            return s_acc + ex

        n_groups = (seg_len + 15) // 16
        s_acc = lax.fori_loop(0, n_groups, g_body, zero16)
        last_off = jnp.maximum(po_j + (n_groups - 1) * 16, 0)
        ex_last = scores_v[pl.ds(last_off, 16)]
        rem = seg_len - (n_groups - 1) * 16
        s_acc = s_acc - jnp.where(lane >= rem, ex_last, 0.0)
        sv = hsum_bcast(s_acc)
        sinv_vec[j, pl.ds(0, 16)] = jnp.where(sv > 0, 1.0 / sv, 0.0)

    def s3_block(g_blk, fbuf, state):
        pos, sj = state
        bs = blk_start(g_blk)
        pe = jnp.minimum(bs + R, range_end)

        def run_body(_, st):
            p, sj_ = st
            a_j = ss_s[sj_]
            run_end = jnp.minimum(ss_s[sj_ + 1], pe)
            po_j = poff_s[sj_]
            sjc = jnp.minimum(sj_, SEG_PER_W - 1)
            sinvb = sinv_vec[sjc, pl.ds(0, 16)]
            accs0 = tuple(racc[sjc, pl.ds(k * 16, 16)] for k in range(NCHUNK))
            delta = po_j - a_j

            def node_quad(t2, accs_in):
                base_n = p + 4 * t2
                rows, abs_ = [], []
                for u in range(4):
                    n_u = base_n + u
                    rows.append(jnp.minimum(n_u, run_end - 1) - bs)
                    if u == 0:
                        aw = scores_v[pl.ds(n_u + delta, 16)]
                    else:
                        idx = jnp.where(n_u < run_end, n_u + delta, SCAP - 16)
                        aw = scores_v[pl.ds(idx, 16)]
                    abs_.append(bcast0(jnp.where(lane == 0, aw, 0.0) * sinvb,
                                       16 + 32 * u))
                return tuple(accs_in[k]
                             + (fbuf[rows[0], pl.ds(k * 16, 16)] * abs_[0]
                                + fbuf[rows[1], pl.ds(k * 16, 16)] * abs_[1])
                             + (fbuf[rows[2], pl.ds(k * 16, 16)] * abs_[2]
                                + fbuf[rows[3], pl.ds(k * 16, 16)] * abs_[3])
                             for k in range(NCHUNK))

            accs1 = lax.fori_loop(0, (run_end - p + 3) // 4, node_quad, accs0)
            for k in range(NCHUNK):
                racc[sjc, pl.ds(k * 16, 16)] = accs1[k]
            adv = ss_s[sj_ + 1] <= pe
            sj_next = jnp.where(adv, jnp.minimum(sj_ + 1, SEG_PER_W), sj_)
            return jnp.maximum(p, run_end), sj_next

        return lax.fori_loop(0, SEG_PER_W, run_body, (pos, sj))

    run_sweep(s3_block, (range_start, jnp.int32(0)))

    pltpu.sync_copy(racc, out_hbm.at[pl.ds(seg0, SEG_PER_W)])


_CALLS = {}


def _get_calls():
    if "attn" not in _CALLS:
        _CALLS["lstm"] = pl.pallas_call(
            _lstm_body,
            out_shape=(jax.ShapeDtypeStruct((B, D), jnp.float32),
                       jax.ShapeDtypeStruct((B, D), jnp.float32)),
        )
        _CALLS["attn"] = functools.partial(
            pl.kernel,
            out_type=jax.ShapeDtypeStruct((B, D), jnp.float32),
            mesh=plsc.VectorSubcoreMesh(core_axis_name="c",
                                        subcore_axis_name="s"),
            compiler_params=pltpu.CompilerParams(needs_layout_passes=False),
            scratch_types=[
                pltpu.VMEM((SCAP,), jnp.float32),
                pltpu.VMEM((R, D), jnp.float32),
                pltpu.VMEM((R, D), jnp.float32),
                pltpu.VMEM((SEG_PER_W, D), jnp.float32),
                pltpu.VMEM((SEG_PER_W, D), jnp.float32),
                pltpu.VMEM((16,), jnp.int32),
                pltpu.VMEM((SEG_PER_W, 16), jnp.float32),
                pltpu.VMEM((SEG_PER_W, 16), jnp.float32),
                pltpu.VMEM((144,), jnp.float32),
                pltpu.VMEM((48,), jnp.float32),
                pltpu.SMEM((16,), jnp.int32),
                pltpu.SMEM((16,), jnp.int32),
                pltpu.SemaphoreType.DMA,
                pltpu.SemaphoreType.DMA,
            ],
        )(_attn_body)
    return _CALLS["lstm"], _CALLS["attn"]


def kernel(feat, segment_ids, W_ih, W_hh, b_ih, b_hh):
    lstm, attn = _get_calls()
    Wq = (W_ih[:, :D] + W_hh).T
    Wr = W_ih[:, D:].T
    bias = (b_ih + b_hh).reshape(1, 4 * D)

    ss = jnp.searchsorted(segment_ids, jnp.arange(B + 1, dtype=jnp.int32)
                          ).astype(jnp.int32)
    ss = jnp.concatenate([ss, jnp.full((7,), N, jnp.int32)])

    h = jnp.zeros((B, D), jnp.float32)
    c = jnp.zeros((B, D), jnp.float32)
    r = jnp.zeros((B, D), jnp.float32)
    for _ in range(N_ITERS):
        h, c = lstm(h, c, r, Wq, Wr, bias)
        r = attn(feat, ss, h)
    return jnp.concatenate([h, r], axis=1)


# speedup vs baseline: 1.2040x; 1.2040x over previous
import functools

import jax
import jax.numpy as jnp
from jax import lax
from jax.experimental import pallas as pl
from jax.experimental.pallas import tpu as pltpu
from jax.experimental.pallas import tpu_sc as plsc

N = 50000
D = 256
B = 256
N_ITERS = 6

NUM_WORKERS = 32
SEG_PER_W = B // NUM_WORKERS
R = 128
SCAP = N + SEG_PER_W * 16 + 16
NCHUNK = D // 16


def _lstm_body(h_ref, c_ref, r_ref, wq_ref, wr_ref, b_ref, h_out, c_out):
    gates = (jnp.dot(h_ref[...], wq_ref[...], preferred_element_type=jnp.float32)
             + jnp.dot(r_ref[...], wr_ref[...], preferred_element_type=jnp.float32)
             + b_ref[...])
    i_g = jax.nn.sigmoid(gates[:, 0:D])
    f_g = jax.nn.sigmoid(gates[:, D:2 * D])
    g_g = jnp.tanh(gates[:, 2 * D:3 * D])
    o_g = jax.nn.sigmoid(gates[:, 3 * D:4 * D])
    c_new = f_g * c_ref[...] + i_g * g_g
    c_out[...] = c_new
    h_out[...] = o_g * jnp.tanh(c_new)


def _attn_body(feat_hbm, ss_hbm, q_hbm, out_hbm,
               scores_v, fbuf, fbuf2, q_own, racc, ss_v, m_vec, sinv_vec,
               hs0, hsm, ss_s, poff_s, sem_a, sem_b):
    wid = lax.axis_index("c") * 16 + lax.axis_index("s")
    seg0 = pl.multiple_of(wid * SEG_PER_W, 8)

    pltpu.sync_copy(ss_hbm.at[pl.ds(seg0, 16)], ss_v)
    pltpu.sync_copy(q_hbm.at[pl.ds(seg0, SEG_PER_W)], q_own)

    ss_vec = ss_v[pl.ds(0, 16)]
    for j in range(SEG_PER_W + 1):
        ss_s[j] = ss_vec[j]
    range_start = ss_s[0]
    range_end = ss_s[SEG_PER_W]
    ss_s[SEG_PER_W + 1] = range_end
    ss_s[SEG_PER_W + 2] = range_end

    zero16 = jnp.zeros((16,), jnp.float32)
    ninf16 = jnp.full((16,), -jnp.inf, jnp.float32)
    lane = lax.iota(jnp.int32, 16)

    for w in range(0, 144, 16):
        hs0[pl.ds(w, 16)] = zero16
    hsm[pl.ds(0, 16)] = ninf16
    hsm[pl.ds(32, 16)] = ninf16
    scores_v[pl.ds(SCAP - 16, 16)] = zero16
    for j in range(SEG_PER_W):
        m_vec[j, pl.ds(0, 16)] = ninf16
        for k in range(NCHUNK):
            racc[j, pl.ds(k * 16, 16)] = zero16


    def hsum_at(v, t, base=16):
        for step in (8, 4, 2, 1):
            hs0[pl.ds(base, 16)] = v
            v = v + hs0[pl.ds(base - step, 16)]
        hs0[pl.ds(base, 16)] = v
        return hs0[pl.ds(base + 15 - t, 16)]

    def bcast0(v, base=16):
        for step in (1, 2, 4, 8):
            hs0[pl.ds(base, 16)] = v
            v = v + hs0[pl.ds(base - step, 16)]
        return v

    def hsum_bcast(v):
        for step in (8, 4, 2, 1):
            hs0[pl.ds(16, 16)] = v
            v = v + hs0[pl.ds(16 - step, 16)]
        v = jnp.where(lane == 15, v, 0.0)
        for step in (1, 2, 4, 8):
            hs0[pl.ds(16, 16)] = v
            v = v + hs0[pl.ds(16 + step, 16)]
        return v

    def hmax_bcast(v):
        for step in (8, 4, 2, 1):
            hsm[pl.ds(16, 16)] = v
            v = jnp.maximum(v, hsm[pl.ds(16 - step, 16)])
        v = jnp.where(lane == 15, v, -jnp.inf)
        for step in (1, 2, 4, 8):
            hsm[pl.ds(16, 16)] = v
            v = jnp.maximum(v, hsm[pl.ds(16 + step, 16)])
        return v

    po = jnp.int32(0)
    for j in range(SEG_PER_W):
        poff_s[j] = po
        seg_len = ss_s[j + 1] - ss_s[j]
        n_groups = (seg_len + 15) // 16
        scores_v[pl.ds(po + jnp.maximum(n_groups - 1, 0) * 16, 16)] = zero16
        po = po + n_groups * 16

    bs0 = (range_start // 8) * 8
    nblk = (range_end - bs0 + R - 1) // R

    def blk_start(bg):
        return pl.multiple_of(
            (jnp.minimum(bs0 + bg * R, N - R) // 8) * 8, 8)

    def dma_start(bg, buf, sem):
        pltpu.make_async_copy(feat_hbm.at[pl.ds(blk_start(bg), R)],
                              buf, sem).start()

    def dma_wait(buf, sem):
        pltpu.make_async_copy(feat_hbm.at[pl.ds(0, R)], buf, sem).wait()

    def run_sweep(process_block, state0):
        dma_start(0, fbuf, sem_a)
        dma_start(1, fbuf2, sem_b)

        def pair_body(gp, st):
            for i, (buf, sem) in enumerate(((fbuf, sem_a), (fbuf2, sem_b))):
                g_blk = gp * 2 + i
                dma_wait(buf, sem)
                st = process_block(g_blk, buf, st)
                dma_start(g_blk + 2, buf, sem)
            return st

        st = lax.fori_loop(0, (nblk + 1) // 2, pair_body, state0)
        dma_wait(fbuf, sem_a)
        dma_wait(fbuf2, sem_b)
        return st

    def s1_block(g_blk, fbuf, state):
        pos, sj = state
        bs = blk_start(g_blk)
        pe = jnp.minimum(bs + R, range_end)

        def run_body(_, st):
            p, sj_ = st
            a_j = ss_s[sj_]
            run_end = jnp.minimum(ss_s[sj_ + 1], pe)
            po_j = poff_s[sj_]
            sjc = jnp.minimum(sj_, SEG_PER_W - 1)

            @pl.when(p < run_end)
            def _():
              qrow = [q_own[sjc, pl.ds(k * 16, 16)] for k in range(NCHUNK)]
              mv0 = m_vec[sjc, pl.ds(0, 16)]

              g0 = (p - a_j) // 16
              g1 = (run_end - a_j + 15) // 16

              def grp_body(g, m_acc):
                  goff = po_j + g * 16
                  gvec0 = scores_v[pl.ds(goff, 16)]
                  gbase = a_j + g * 16
                  lo = jnp.maximum(p, gbase)
                  hi = jnp.minimum(run_end, gbase + 16)

                  def node_pair(t2, gvec):
                      na = lo + 2 * t2
                      nb = jnp.minimum(na + 1, hi - 1)

                      def dot(n, base):
                          row = n - bs
                          acc = [fbuf[row, pl.ds(k * 16, 16)] * qrow[k]
                                 for k in range(4)]
                          for k in range(4, NCHUNK):
                              acc[k % 4] = (acc[k % 4] + fbuf[row, pl.ds(k * 16, 16)]
                                            * qrow[k])
                          return hsum_at((acc[0] + acc[1]) + (acc[2] + acc[3]),
                                         n - gbase, base)

                      sba = dot(na, 16)
                      sbb = dot(nb, 48)
                      gvec = jnp.where(lane == (na - gbase), sba, gvec)
                      return jnp.where(lane == (nb - gbase), sbb, gvec)

                  gvec1 = lax.fori_loop(0, (hi - lo + 1) // 2, node_pair, gvec0)
                  scores_v[pl.ds(goff, 16)] = gvec1
                  valid = lane < (hi - gbase)
                  return jnp.maximum(m_acc, jnp.where(valid, gvec1, ninf16))

              m_fin = lax.fori_loop(g0, g1, grp_body, mv0)
              m_vec[sjc, pl.ds(0, 16)] = m_fin

            adv = ss_s[sj_ + 1] <= pe
            sj_next = jnp.where(adv, jnp.minimum(sj_ + 1, SEG_PER_W), sj_)
            return jnp.maximum(p, run_end), sj_next

        return lax.fori_loop(0, SEG_PER_W, run_body, (pos, sj))

    run_sweep(s1_block, (range_start, jnp.int32(0)))

    for j in range(SEG_PER_W):
        seg_len = ss_s[j + 1] - ss_s[j]
        po_j = poff_s[j]
        mb = hmax_bcast(m_vec[j, pl.ds(0, 16)])

        def g_body(g, s_acc, po_j=po_j, mb=mb):
            off = po_j + g * 16
            ex = jnp.exp(jnp.minimum(scores_v[pl.ds(off, 16)] - mb, 0.0))
            scores_v[pl.ds(off, 16)] = ex
            return s_acc + ex

        n_groups = (seg_len + 15) // 16
        s_acc = lax.fori_loop(0, n_groups, g_body, zero16)
        last_off = jnp.maximum(po_j + (n_groups - 1) * 16, 0)
        ex_last = scores_v[pl.ds(last_off, 16)]
        rem = seg_len - (n_groups - 1) * 16
        s_acc = s_acc - jnp.where(lane >= rem, ex_last, 0.0)
        sv = hsum_bcast(s_acc)
        sinv_vec[j, pl.ds(0, 16)] = jnp.where(sv > 0, 1.0 / sv, 0.0)

    def s3_block(g_blk, fbuf, state):
        pos, sj = state
        bs = blk_start(g_blk)
        pe = jnp.minimum(bs + R, range_end)

        def run_body(_, st):
            p, sj_ = st
            a_j = ss_s[sj_]
            run_end = jnp.minimum(ss_s[sj_ + 1], pe)
            po_j = poff_s[sj_]
            sjc = jnp.minimum(sj_, SEG_PER_W - 1)
            delta = po_j - a_j

            @pl.when(p < run_end)
            def _():
              sinvb = sinv_vec[sjc, pl.ds(0, 16)]
              accs0 = tuple(racc[sjc, pl.ds(k * 16, 16)] for k in range(NCHUNK))

              def node_pair(t2, accs_in):
                  na = p + 2 * t2
                  nb = na + 1
                  rowa = na - bs
                  rowb = jnp.minimum(nb, run_end - 1) - bs
                  awa = scores_v[pl.ds(na + delta, 16)]
                  idx_b = jnp.where(nb < run_end, nb + delta, SCAP - 16)
                  awb = scores_v[pl.ds(idx_b, 16)]
                  aba = bcast0(jnp.where(lane == 0, awa, 0.0) * sinvb, 16)
                  abb = bcast0(jnp.where(lane == 0, awb, 0.0) * sinvb, 48)
                  return tuple(accs_in[k]
                               + fbuf[rowa, pl.ds(k * 16, 16)] * aba
                               + fbuf[rowb, pl.ds(k * 16, 16)] * abb
                               for k in range(NCHUNK))

              accs1 = lax.fori_loop(0, (run_end - p + 1) // 2, node_pair, accs0)
              for k in range(NCHUNK):
                  racc[sjc, pl.ds(k * 16, 16)] = accs1[k]

            adv = ss_s[sj_ + 1] <= pe
            sj_next = jnp.where(adv, jnp.minimum(sj_ + 1, SEG_PER_W), sj_)
            return jnp.maximum(p, run_end), sj_next

        return lax.fori_loop(0, SEG_PER_W, run_body, (pos, sj))

    run_sweep(s3_block, (range_start, jnp.int32(0)))

    pltpu.sync_copy(racc, out_hbm.at[pl.ds(seg0, SEG_PER_W)])


_CALLS = {}


def _get_calls():
    if "attn" not in _CALLS:
        _CALLS["lstm"] = pl.pallas_call(
            _lstm_body,
            out_shape=(jax.ShapeDtypeStruct((B, D), jnp.float32),
                       jax.ShapeDtypeStruct((B, D), jnp.float32)),
        )
        _CALLS["attn"] = functools.partial(
            pl.kernel,
            out_type=jax.ShapeDtypeStruct((B, D), jnp.float32),
            mesh=plsc.VectorSubcoreMesh(core_axis_name="c",
                                        subcore_axis_name="s"),
            compiler_params=pltpu.CompilerParams(needs_layout_passes=False),
            scratch_types=[
                pltpu.VMEM((SCAP,), jnp.float32),
                pltpu.VMEM((R, D), jnp.float32),
                pltpu.VMEM((R, D), jnp.float32),
                pltpu.VMEM((SEG_PER_W, D), jnp.float32),
                pltpu.VMEM((SEG_PER_W, D), jnp.float32),
                pltpu.VMEM((16,), jnp.int32),
                pltpu.VMEM((SEG_PER_W, 16), jnp.float32),
                pltpu.VMEM((SEG_PER_W, 16), jnp.float32),
                pltpu.VMEM((144,), jnp.float32),
                pltpu.VMEM((48,), jnp.float32),
                pltpu.SMEM((16,), jnp.int32),
                pltpu.SMEM((16,), jnp.int32),
                pltpu.SemaphoreType.DMA,
                pltpu.SemaphoreType.DMA,
            ],
        )(_attn_body)
    return _CALLS["lstm"], _CALLS["attn"]


def kernel(feat, segment_ids, W_ih, W_hh, b_ih, b_hh):
    lstm, attn = _get_calls()
    Wq = (W_ih[:, :D] + W_hh).T
    Wr = W_ih[:, D:].T
    bias = (b_ih + b_hh).reshape(1, 4 * D)

    ss = jnp.searchsorted(segment_ids, jnp.arange(B + 1, dtype=jnp.int32)
                          ).astype(jnp.int32)
    ss = jnp.concatenate([ss, jnp.full((7,), N, jnp.int32)])

    h = jnp.zeros((B, D), jnp.float32)
    c = jnp.zeros((B, D), jnp.float32)
    r = jnp.zeros((B, D), jnp.float32)
    for _ in range(N_ITERS):
        h, c = lstm(h, c, r, Wq, Wr, bias)
        r = attn(feat, ss, h)
    return jnp.concatenate([h, r], axis=1)


# speedup vs baseline: 1.2142x; 1.0084x over previous
import functools

import jax
import jax.numpy as jnp
from jax import lax
from jax.experimental import pallas as pl
from jax.experimental.pallas import tpu as pltpu
from jax.experimental.pallas import tpu_sc as plsc

N = 50000
D = 256
B = 256
N_ITERS = 6

NUM_WORKERS = 32
SEG_PER_W = B // NUM_WORKERS
R = 144
SCAP = N + SEG_PER_W * 16 + 16
NCHUNK = D // 16


def _lstm_body(h_ref, c_ref, r_ref, wq_ref, wr_ref, b_ref, h_out, c_out):
    gates = (jnp.dot(h_ref[...], wq_ref[...], preferred_element_type=jnp.float32)
             + jnp.dot(r_ref[...], wr_ref[...], preferred_element_type=jnp.float32)
             + b_ref[...])
    i_g = jax.nn.sigmoid(gates[:, 0:D])
    f_g = jax.nn.sigmoid(gates[:, D:2 * D])
    g_g = jnp.tanh(gates[:, 2 * D:3 * D])
    o_g = jax.nn.sigmoid(gates[:, 3 * D:4 * D])
    c_new = f_g * c_ref[...] + i_g * g_g
    c_out[...] = c_new
    h_out[...] = o_g * jnp.tanh(c_new)


def _attn_body(feat_hbm, ss_hbm, q_hbm, out_hbm,
               scores_v, fbuf, fbuf2, q_own, racc, ss_v, m_vec, sinv_vec,
               hs0, hsm, ss_s, poff_s, sem_a, sem_b):
    wid = lax.axis_index("c") * 16 + lax.axis_index("s")
    seg0 = pl.multiple_of(wid * SEG_PER_W, 8)

    pltpu.sync_copy(ss_hbm.at[pl.ds(seg0, 16)], ss_v)
    pltpu.sync_copy(q_hbm.at[pl.ds(seg0, SEG_PER_W)], q_own)

    ss_vec = ss_v[pl.ds(0, 16)]
    for j in range(SEG_PER_W + 1):
        ss_s[j] = ss_vec[j]
    range_start = ss_s[0]
    range_end = ss_s[SEG_PER_W]
    ss_s[SEG_PER_W + 1] = range_end
    ss_s[SEG_PER_W + 2] = range_end

    zero16 = jnp.zeros((16,), jnp.float32)
    ninf16 = jnp.full((16,), -jnp.inf, jnp.float32)
    lane = lax.iota(jnp.int32, 16)

    for w in range(0, 144, 16):
        hs0[pl.ds(w, 16)] = zero16
    hsm[pl.ds(0, 16)] = ninf16
    hsm[pl.ds(32, 16)] = ninf16
    scores_v[pl.ds(SCAP - 16, 16)] = zero16
    for j in range(SEG_PER_W):
        m_vec[j, pl.ds(0, 16)] = ninf16
        for k in range(NCHUNK):
            racc[j, pl.ds(k * 16, 16)] = zero16


    def hsum_at(v, t, base=16):
        for step in (8, 4, 2, 1):
            hs0[pl.ds(base, 16)] = v
            v = v + hs0[pl.ds(base - step, 16)]
        hs0[pl.ds(base, 16)] = v
        return hs0[pl.ds(base + 15 - t, 16)]

    def bcast0(v, base=16):
        for step in (1, 2, 4, 8):
            hs0[pl.ds(base, 16)] = v
            v = v + hs0[pl.ds(base - step, 16)]
        return v

    def hsum_bcast(v):
        for step in (8, 4, 2, 1):
            hs0[pl.ds(16, 16)] = v
            v = v + hs0[pl.ds(16 - step, 16)]
        v = jnp.where(lane == 15, v, 0.0)
        for step in (1, 2, 4, 8):
            hs0[pl.ds(16, 16)] = v
            v = v + hs0[pl.ds(16 + step, 16)]
        return v

    def hmax_bcast(v):
        for step in (8, 4, 2, 1):
            hsm[pl.ds(16, 16)] = v
            v = jnp.maximum(v, hsm[pl.ds(16 - step, 16)])
        v = jnp.where(lane == 15, v, -jnp.inf)
        for step in (1, 2, 4, 8):
            hsm[pl.ds(16, 16)] = v
            v = jnp.maximum(v, hsm[pl.ds(16 + step, 16)])
        return v

    po = jnp.int32(0)
    for j in range(SEG_PER_W):
        poff_s[j] = po
        seg_len = ss_s[j + 1] - ss_s[j]
        n_groups = (seg_len + 15) // 16
        scores_v[pl.ds(po + jnp.maximum(n_groups - 1, 0) * 16, 16)] = zero16
        po = po + n_groups * 16

    bs0 = (range_start // 8) * 8
    nblk = (range_end - bs0 + R - 1) // R

    def blk_start(bg):
        return pl.multiple_of(
            (jnp.minimum(bs0 + bg * R, N - R) // 8) * 8, 8)

    def dma_start(bg, buf, sem):
        pltpu.make_async_copy(feat_hbm.at[pl.ds(blk_start(bg), R)],
                              buf, sem).start()

    def dma_wait(buf, sem):
        pltpu.make_async_copy(feat_hbm.at[pl.ds(0, R)], buf, sem).wait()

    def run_sweep(process_block, state0):
        dma_start(0, fbuf, sem_a)
        dma_start(1, fbuf2, sem_b)

        def pair_body(gp, st):
            for i, (buf, sem) in enumerate(((fbuf, sem_a), (fbuf2, sem_b))):
                g_blk = gp * 2 + i
                dma_wait(buf, sem)
                st = process_block(g_blk, buf, st)
                dma_start(g_blk + 2, buf, sem)
            return st

        st = lax.fori_loop(0, (nblk + 1) // 2, pair_body, state0)
        dma_wait(fbuf, sem_a)
        dma_wait(fbuf2, sem_b)
        return st

    def s1_block(g_blk, fbuf, state):
        pos, sj = state
        bs = blk_start(g_blk)
        pe = jnp.minimum(bs + R, range_end)

        def run_body(_, st):
            p, sj_ = st
            a_j = ss_s[sj_]
            run_end = jnp.minimum(ss_s[sj_ + 1], pe)
            po_j = poff_s[sj_]
            sjc = jnp.minimum(sj_, SEG_PER_W - 1)

            @pl.when(p < run_end)
            def _():
              qrow = [q_own[sjc, pl.ds(k * 16, 16)] for k in range(NCHUNK)]
              mv0 = m_vec[sjc, pl.ds(0, 16)]

              g0 = (p - a_j) // 16
              g1 = (run_end - a_j + 15) // 16

              def grp_body(g, m_acc):
                  goff = po_j + g * 16
                  gvec0 = scores_v[pl.ds(goff, 16)]
                  gbase = a_j + g * 16
                  lo = jnp.maximum(p, gbase)
                  hi = jnp.minimum(run_end, gbase + 16)

                  def node_pair(t2, gvec):
                      na = lo + 2 * t2
                      nb = jnp.minimum(na + 1, hi - 1)

                      def dot(n, base):
                          row = n - bs
                          acc = [fbuf[row, pl.ds(k * 16, 16)] * qrow[k]
                                 for k in range(4)]
                          for k in range(4, NCHUNK):
                              acc[k % 4] = (acc[k % 4] + fbuf[row, pl.ds(k * 16, 16)]
                                            * qrow[k])
                          return hsum_at((acc[0] + acc[1]) + (acc[2] + acc[3]),
                                         n - gbase, base)

                      sba = dot(na, 16)
                      sbb = dot(nb, 48)
                      gvec = jnp.where(lane == (na - gbase), sba, gvec)
                      return jnp.where(lane == (nb - gbase), sbb, gvec)

                  gvec1 = lax.fori_loop(0, (hi - lo + 1) // 2, node_pair, gvec0)
                  scores_v[pl.ds(goff, 16)] = gvec1
                  valid = lane < (hi - gbase)
                  return jnp.maximum(m_acc, jnp.where(valid, gvec1, ninf16))

              m_fin = lax.fori_loop(g0, g1, grp_body, mv0)
              m_vec[sjc, pl.ds(0, 16)] = m_fin

            adv = ss_s[sj_ + 1] <= pe
            sj_next = jnp.where(adv, jnp.minimum(sj_ + 1, SEG_PER_W), sj_)
            return jnp.maximum(p, run_end), sj_next

        return lax.fori_loop(0, SEG_PER_W, run_body, (pos, sj))

    run_sweep(s1_block, (range_start, jnp.int32(0)))

    for j in range(SEG_PER_W):
        seg_len = ss_s[j + 1] - ss_s[j]
        po_j = poff_s[j]
        mb = hmax_bcast(m_vec[j, pl.ds(0, 16)])

        def g_body(g, s_acc, po_j=po_j, mb=mb):
            off = po_j + g * 16
            ex = jnp.exp(jnp.minimum(scores_v[pl.ds(off, 16)] - mb, 0.0))
            scores_v[pl.ds(off, 16)] = ex
            return s_acc + ex

        n_groups = (seg_len + 15) // 16
        s_acc = lax.fori_loop(0, n_groups, g_body, zero16)
        last_off = jnp.maximum(po_j + (n_groups - 1) * 16, 0)
        ex_last = scores_v[pl.ds(last_off, 16)]
        rem = seg_len - (n_groups - 1) * 16
        s_acc = s_acc - jnp.where(lane >= rem, ex_last, 0.0)
        sv = hsum_bcast(s_acc)
        sinv_vec[j, pl.ds(0, 16)] = jnp.where(sv > 0, 1.0 / sv, 0.0)

    def s3_block(g_blk, fbuf, state):
        pos, sj = state
        bs = blk_start(g_blk)
        pe = jnp.minimum(bs + R, range_end)

        def run_body(_, st):
            p, sj_ = st
            a_j = ss_s[sj_]
            run_end = jnp.minimum(ss_s[sj_ + 1], pe)
            po_j = poff_s[sj_]
            sjc = jnp.minimum(sj_, SEG_PER_W - 1)
            delta = po_j - a_j

            @pl.when(p < run_end)
            def _():
              sinvb = sinv_vec[sjc, pl.ds(0, 16)]
              accs0 = tuple(racc[sjc, pl.ds(k * 16, 16)] for k in range(NCHUNK))

              def node_pair(t2, accs_in):
                  na = p + 2 * t2
                  nb = na + 1
                  rowa = na - bs
                  rowb = jnp.minimum(nb, run_end - 1) - bs
                  awa = scores_v[pl.ds(na + delta, 16)]
                  idx_b = jnp.where(nb < run_end, nb + delta, SCAP - 16)
                  awb = scores_v[pl.ds(idx_b, 16)]
                  aba = bcast0(jnp.where(lane == 0, awa, 0.0) * sinvb, 16)
                  abb = bcast0(jnp.where(lane == 0, awb, 0.0) * sinvb, 48)
                  return tuple(accs_in[k]
                               + fbuf[rowa, pl.ds(k * 16, 16)] * aba
                               + fbuf[rowb, pl.ds(k * 16, 16)] * abb
                               for k in range(NCHUNK))

              accs1 = lax.fori_loop(0, (run_end - p + 1) // 2, node_pair, accs0)
              for k in range(NCHUNK):
                  racc[sjc, pl.ds(k * 16, 16)] = accs1[k]

            adv = ss_s[sj_ + 1] <= pe
            sj_next = jnp.where(adv, jnp.minimum(sj_ + 1, SEG_PER_W), sj_)
            return jnp.maximum(p, run_end), sj_next

        return lax.fori_loop(0, SEG_PER_W, run_body, (pos, sj))

    run_sweep(s3_block, (range_start, jnp.int32(0)))

    pltpu.sync_copy(racc, out_hbm.at[pl.ds(seg0, SEG_PER_W)])


_CALLS = {}


def _get_calls():
    if "attn" not in _CALLS:
        _CALLS["lstm"] = pl.pallas_call(
            _lstm_body,
            out_shape=(jax.ShapeDtypeStruct((B, D), jnp.float32),
                       jax.ShapeDtypeStruct((B, D), jnp.float32)),
        )
        _CALLS["attn"] = functools.partial(
            pl.kernel,
            out_type=jax.ShapeDtypeStruct((B, D), jnp.float32),
            mesh=plsc.VectorSubcoreMesh(core_axis_name="c",
                                        subcore_axis_name="s"),
            compiler_params=pltpu.CompilerParams(needs_layout_passes=False),
            scratch_types=[
                pltpu.VMEM((SCAP,), jnp.float32),
                pltpu.VMEM((R, D), jnp.float32),
                pltpu.VMEM((R, D), jnp.float32),
                pltpu.VMEM((SEG_PER_W, D), jnp.float32),
                pltpu.VMEM((SEG_PER_W, D), jnp.float32),
                pltpu.VMEM((16,), jnp.int32),
                pltpu.VMEM((SEG_PER_W, 16), jnp.float32),
                pltpu.VMEM((SEG_PER_W, 16), jnp.float32),
                pltpu.VMEM((144,), jnp.float32),
                pltpu.VMEM((48,), jnp.float32),
                pltpu.SMEM((16,), jnp.int32),
                pltpu.SMEM((16,), jnp.int32),
                pltpu.SemaphoreType.DMA,
                pltpu.SemaphoreType.DMA,
            ],
        )(_attn_body)
    return _CALLS["lstm"], _CALLS["attn"]


def kernel(feat, segment_ids, W_ih, W_hh, b_ih, b_hh):
    lstm, attn = _get_calls()
    Wq = (W_ih[:, :D] + W_hh).T
    Wr = W_ih[:, D:].T
    bias = (b_ih + b_hh).reshape(1, 4 * D)

    ss = jnp.searchsorted(segment_ids, jnp.arange(B + 1, dtype=jnp.int32)
                          ).astype(jnp.int32)
    ss = jnp.concatenate([ss, jnp.full((7,), N, jnp.int32)])

    h = jnp.zeros((B, D), jnp.float32)
    c = jnp.zeros((B, D), jnp.float32)
    r = jnp.zeros((B, D), jnp.float32)
    for _ in range(N_ITERS):
        h, c = lstm(h, c, r, Wq, Wr, bias)
        r = attn(feat, ss, h)
    return jnp.concatenate([h, r], axis=1)
